# baseline (speedup 1.0000x reference)
import functools

import jax
import jax.numpy as jnp
from jax import lax
from jax.experimental import pallas as pl
from jax.experimental.pallas import tpu as pltpu
from jax.experimental.pallas import tpu_sc as plsc

B = 16384
K = 21
DIM = 10
V = 1000000

NW = 32
SPT = B // NW
CS = 128
NCHUNK = SPT // CS
BW = 8
NBLK = V * DIM // BW


def _build_kernel():
    mesh = plsc.VectorSubcoreMesh(core_axis_name="c", subcore_axis_name="s")

    @functools.partial(
        pl.kernel,
        mesh=mesh,
        compiler_params=pltpu.CompilerParams(needs_layout_passes=False,
                                             use_tc_tiling_on_sc=False),
        out_type=jax.ShapeDtypeStruct((B * K,), jnp.float32),
        scratch_types=[
            pltpu.VMEM((SPT,), jnp.int32),
            pltpu.VMEM((DIM * NCHUNK, CS), jnp.int32),
            pltpu.VMEM((DIM * NCHUNK, CS), jnp.float32),
            pltpu.VMEM((K, CS), jnp.int32),
            pltpu.VMEM((2 * K, CS), jnp.int32),
            pltpu.VMEM((2, K, CS, BW), jnp.float32),
            pltpu.VMEM((SPT * K,), jnp.float32),
            pltpu.SemaphoreType.DMA,
            pltpu.SemaphoreType.DMA,
        ],
    )
    def sc_kernel(idx_in_hbm, idx_out_hbm, win_hbm, wout_hbm, out_hbm,
                  idx_in_v, in_idx_v, in_vals_v, oidx_v, qidx_v, blk_v, out_v,
                  sem_in, sem_rows):
        wid = lax.axis_index("c") * 16 + lax.axis_index("s")
        iota = lax.iota(jnp.int32, 16)

        pltpu.sync_copy(idx_in_hbm.at[pl.ds(wid * SPT, SPT)], idx_in_v)

        def build_in_idx(j, carry):
            base = (j >> 2) * V
            part = j & 3
            for r in range(CS // 16):
                v = idx_in_v[pl.ds(part * CS + r * 16, 16)]
                in_idx_v[j, pl.ds(r * 16, 16)] = v + base
            return carry

        lax.fori_loop(0, DIM * NCHUNK, build_in_idx, 0)

        def fire_in(j, carry):
            pltpu.async_copy(win_hbm.at[in_idx_v.at[j]], in_vals_v.at[j], sem_in)
            return carry

        lax.fori_loop(0, DIM * NCHUNK, fire_in, 0)

        def drain_in(j, carry):
            pltpu.make_async_copy(win_hbm.at[in_idx_v.at[j]],
                                  in_vals_v.at[j], sem_in).wait()
            return carry

        lax.fori_loop(0, DIM * NCHUNK, drain_in, 0)

        for c in range(NCHUNK):
            row0 = (wid * NCHUNK + c) * K
            pltpu.sync_copy(idx_out_hbm.at[pl.ds(row0, K)], oidx_v)

            def build_q(j, carry):
                for r in range(CS // 16):
                    v = oidx_v[j, pl.ds(r * 16, 16)]
                    p0 = (v * DIM) >> 3
                    qidx_v[j, pl.ds(r * 16, 16)] = p0
                    qidx_v[j + K, pl.ds(r * 16, 16)] = p0 + 1
                return carry

            lax.fori_loop(0, K, build_q, 0)

            handles = []
            for j in range(K):
                handles.append(pltpu.async_copy(
                    wout_hbm.at[qidx_v.at[j]], blk_v.at[0, j], sem_rows))
                handles.append(pltpu.async_copy(
                    wout_hbm.at[qidx_v.at[j + K]], blk_v.at[1, j], sem_rows))
            for h in handles:
                h.wait()

            def grp(g, carry, c=c):
                sbase = g * 16
                i1_in = sbase + iota
                ivs = [
                    plsc.load_gather(
                        in_vals_v,
                        [jnp.full((16,), d * NCHUNK + c, jnp.int32), i1_in])
                    for d in range(DIM)
                ]
                for k in range(K):
                    r = iota * K + (g * (16 * K) + k)
                    jv = r >> 7
                    iv = r & (CS - 1)
                    idxv = plsc.load_gather(oidx_v, [jv, iv])
                    w0 = (idxv * DIM) & 7
                    acc = jnp.zeros((16,), jnp.float32)
                    for d in range(DIM):
                        t = w0 + d
                        sel = t >> 3
                        wv = t & (BW - 1)
                        vv = plsc.load_gather(blk_v, [sel, jv, iv, wv])
                        acc = acc + vv * ivs[d]
                    plsc.store_scatter(out_v, [r + c * (K * CS)], acc)
                return carry

            lax.fori_loop(0, CS // 16, grp, 0)

        pltpu.sync_copy(out_v, out_hbm.at[pl.ds(wid * (SPT * K), SPT * K)])

    return sc_kernel


_SC_KERNEL = _build_kernel()


@jax.jit
def kernel(input_index_batch, output_indices_batch, W_in, W_out):
    idx_in1d = input_index_batch.astype(jnp.int32).reshape(B)
    idx_out2d = output_indices_batch.astype(jnp.int32).reshape(B * K // CS, CS)
    win_flat = W_in.reshape(DIM * V)
    wout_blk = W_out.reshape(NBLK, BW)
    out = _SC_KERNEL(idx_in1d, idx_out2d, win_flat, wout_blk)
    return out.reshape(B, K)
